# baseline (device time: 57945 ns/iter reference)
import numpy as np
import jax
import jax.numpy as jnp
from jax import lax
from jax.experimental import pallas as pl
from jax.experimental.pallas import tpu as pltpu

N_DEV = 4
B = 2
SQ_G = 1024
SQ_L = SQ_G // N_DEV
SQ_H = SQ_L // 2
D_MODEL = 768
H_L = 4
DH = 64
HD_L = H_L * DH
SCALE = 0.125
BF = jnp.bfloat16
F32 = jnp.float32


def _rope_tables():
    inv = 1.0 / (10000.0 ** (np.arange(0, DH, 2) / DH))
    pos = np.arange(SQ_G)[:, None] * inv[None, :]
    cos = np.repeat(np.cos(pos), 2, axis=-1)
    sin = np.repeat(np.sin(pos), 2, axis=-1)
    cos_t = np.tile(cos, (1, H_L)).astype(np.float32)
    sin_t = np.tile(sin, (1, H_L)).astype(np.float32)
    p64 = np.zeros((DH, DH), dtype=np.float32)
    for k in range(DH // 2):
        p64[2 * k + 1, 2 * k] = -1.0
        p64[2 * k, 2 * k + 1] = 1.0
    perm = np.kron(np.eye(H_L, dtype=np.float32), p64)
    return (jnp.asarray(cos_t), jnp.asarray(sin_t),
            jnp.asarray(perm, dtype=BF))


def kernel(x, Wq, Wk, Wv, Wo):
    cos_t, sin_t, perm = _rope_tables()

    def body(x_ref, wq_ref, wk_ref, wv_ref, wo_ref, cos_ref, sin_ref, p_ref,
             out_ref,
             xg_ref, q_ref, k_ref, v_ref,
             sbl_ref, sbr_ref, sbd_ref, rfl_ref, rfr_ref, rfd_ref,
             agr_send, agr_recv, agl_send, agl_recv,
             sl_send, fr_recv, sr_send, fl_recv, sd_send, fd_recv):
        me = lax.axis_index("i")
        left = (me - 1) % N_DEV
        right = (me + 1) % N_DEV
        diag = (me + 2) % N_DEV

        barrier = pltpu.get_barrier_semaphore()
        for nbr in (left, right, diag):
            pl.semaphore_signal(
                barrier, inc=1,
                device_id=(nbr,), device_id_type=pl.DeviceIdType.MESH,
            )
        pl.semaphore_wait(barrier, 3)

        def rdma(src, dst, ssem, rsem, to):
            return pltpu.make_async_remote_copy(
                src_ref=src, dst_ref=dst, send_sem=ssem, recv_sem=rsem,
                device_id=(to,), device_id_type=pl.DeviceIdType.MESH,
            )

        def xg_full(c):
            return xg_ref.at[:, pl.ds(c * SQ_L, SQ_L), :]

        def xg_half(c, which):
            return xg_ref.at[:, pl.ds(c * SQ_L + which * SQ_H, SQ_H), :]

        wq_bf = wq_ref[...].astype(BF)
        wk_bf = wk_ref[...].astype(BF)
        wv_bf = wv_ref[...].astype(BF)
        wo_bf = wo_ref[...].astype(BF)

        def qkv_chunk(c):
            rows = pl.ds(c * SQ_L, SQ_L)
            cos = cos_ref[rows, :]
            sin = sin_ref[rows, :]
            p = p_ref[...]
            for b in range(B):
                xb = xg_ref[b, rows, :]
                q = jnp.dot(xb, wq_bf, preferred_element_type=F32)
                k = jnp.dot(xb, wk_bf, preferred_element_type=F32)
                qr = jnp.dot(q.astype(BF), p, preferred_element_type=F32)
                kr = jnp.dot(k.astype(BF), p, preferred_element_type=F32)
                q_ref[b, rows, :] = ((q * cos + qr * sin) * SCALE).astype(BF)
                k_ref[b, rows, :] = (k * cos + kr * sin).astype(BF)
                v_ref[b, rows, :] = jnp.dot(
                    xb, wv_bf, preferred_element_type=F32).astype(BF)

        acc = {}
        chunk_of = {"me": me, "left": left, "right": right, "diag": diag}

        def upd(qrole, krole):
            qrows = pl.ds(chunk_of[qrole] * SQ_L, SQ_L)
            krows = pl.ds(chunk_of[krole] * SQ_L, SQ_L)
            st = acc.setdefault(qrole, {})
            for b in range(B):
                for h in range(H_L):
                    hsl = slice(h * DH, (h + 1) * DH)
                    qh = q_ref[b, qrows, hsl]
                    kh = k_ref[b, krows, hsl]
                    vh = v_ref[b, krows, hsl]
                    e = jnp.exp(lax.dot_general(
                        qh, kh, (((1,), (1,)), ((), ())),
                        preferred_element_type=F32))
                    d = jnp.sum(e, axis=-1, keepdims=True)
                    c = jnp.dot(e.astype(BF), vh, preferred_element_type=F32)
                    if (b, h) in st:
                        c0, d0 = st[(b, h)]
                        st[(b, h)] = (c0 + c, d0 + d)
                    else:
                        st[(b, h)] = (c, d)

        def finalize(qrole):
            st = acc[qrole]
            outs = []
            for b in range(B):
                ctxs = []
                for h in range(H_L):
                    c, d = st[(b, h)]
                    ctxs.append(c * (1.0 / d))
                ctx = jnp.concatenate(ctxs, axis=-1).astype(BF)
                outs.append(jnp.dot(ctx, wo_bf, preferred_element_type=F32))
            return outs

        xg_ref[:, pl.ds(me * SQ_L, SQ_L), :] = x_ref[...].astype(BF)
        r0 = rdma(xg_full(me), xg_full(me), agr_send.at[0],
                  agr_recv.at[0], right)
        r0.start()
        l0 = rdma(xg_full(me), xg_full(me), agl_send.at[0],
                  agl_recv.at[0], left)
        l0.start()
        qkv_chunk(me)
        upd("me", "me")

        r0.wait_recv()
        r1 = rdma(xg_half(left, 0), xg_half(left, 0), agr_send.at[1],
                  agr_recv.at[1], right)
        r1.start()
        qkv_chunk(left)

        l0.wait_recv()
        l1 = rdma(xg_half(right, 1), xg_half(right, 1), agl_send.at[1],
                  agl_recv.at[1], left)
        l1.start()
        qkv_chunk(right)

        upd("me", "left")
        upd("left", "left")
        upd("left", "me")
        upd("me", "right")
        upd("right", "right")
        upd("right", "me")
        upd("left", "right")
        upd("right", "left")

        r1.wait_recv()
        l1.wait_recv()
        qkv_chunk(diag)

        upd("left", "diag")
        for b, pb in enumerate(finalize("left")):
            sbl_ref[b] = pb.astype(BF)
        dl = rdma(sbl_ref, rfr_ref, sl_send, fr_recv, left)
        dl.start()

        upd("right", "diag")
        for b, pb in enumerate(finalize("right")):
            sbr_ref[b] = pb.astype(BF)
        dr = rdma(sbr_ref, rfl_ref, sr_send, fl_recv, right)
        dr.start()

        upd("diag", "me")
        upd("diag", "left")
        upd("diag", "right")
        upd("diag", "diag")
        for b, pb in enumerate(finalize("diag")):
            sbd_ref[b] = pb.astype(BF)
        dd = rdma(sbd_ref, rfd_ref, sd_send, fd_recv, diag)
        dd.start()

        upd("me", "diag")
        p_own = finalize("me")

        dl.wait_recv()
        dr.wait_recv()
        dd.wait_recv()
        for b in range(B):
            out_ref[b] = (p_own[b] + rfr_ref[b].astype(F32)
                          + rfl_ref[b].astype(F32) + rfd_ref[b].astype(F32))

        for d_ in (r0, r1, l0, l1, dl, dr, dd):
            d_.wait_send()

    return pl.pallas_call(
        body,
        out_shape=jax.ShapeDtypeStruct((B, SQ_L, D_MODEL), jnp.float32),
        in_specs=[pl.BlockSpec(memory_space=pltpu.VMEM)] * 8,
        out_specs=pl.BlockSpec(memory_space=pltpu.VMEM),
        scratch_shapes=[
            pltpu.VMEM((B, SQ_G, D_MODEL), BF),
            pltpu.VMEM((B, SQ_G, HD_L), BF),
            pltpu.VMEM((B, SQ_G, HD_L), BF),
            pltpu.VMEM((B, SQ_G, HD_L), BF),
            pltpu.VMEM((B, SQ_L, D_MODEL), BF),
            pltpu.VMEM((B, SQ_L, D_MODEL), BF),
            pltpu.VMEM((B, SQ_L, D_MODEL), BF),
            pltpu.VMEM((B, SQ_L, D_MODEL), BF),
            pltpu.VMEM((B, SQ_L, D_MODEL), BF),
            pltpu.VMEM((B, SQ_L, D_MODEL), BF),
            pltpu.SemaphoreType.DMA((2,)),
            pltpu.SemaphoreType.DMA((2,)),
            pltpu.SemaphoreType.DMA((2,)),
            pltpu.SemaphoreType.DMA((2,)),
            pltpu.SemaphoreType.DMA(()),
            pltpu.SemaphoreType.DMA(()),
            pltpu.SemaphoreType.DMA(()),
            pltpu.SemaphoreType.DMA(()),
            pltpu.SemaphoreType.DMA(()),
            pltpu.SemaphoreType.DMA(()),
        ],
        compiler_params=pltpu.CompilerParams(collective_id=0),
    )(x, Wq, Wk, Wv, Wo, cos_t, sin_t, perm)
